# baseline (speedup 1.0000x reference)
import functools

import jax
import jax.numpy as jnp
from jax import lax
from jax.experimental import pallas as pl
from jax.experimental.pallas import tpu as pltpu
from jax.experimental.pallas import tpu_sc as plsc

VOCAB = 100000
EMBED_DIM = 16
BATCH = 16384

_NC = 2
_NS = 16
_NW = _NC * _NS
_LANES = 16

_CHUNK = 128
_B_PER_W = BATCH // _NW
_N_CHUNKS = _B_PER_W // _CHUNK
_STRIDE = VOCAB + 8


def _lookup_kernel(idx_hbm, tab_hbm, out_hbm, idx_v, dst_v, sem):
    wid = lax.axis_index("s") * _NC + lax.axis_index("c")
    base = wid * _B_PER_W

    pltpu.sync_copy(idx_hbm.at[pl.ds(base, _B_PER_W)], idx_v)

    def remap(i, carry):
        sl = pl.ds(i * _LANES, _LANES)
        v = idx_v[sl]
        idx_v[sl] = jnp.where((v >= 0) & (v < VOCAB), v + 1, 0)
        return carry

    lax.fori_loop(0, _B_PER_W // _LANES, remap, 0)

    def do_chunk(c, carry):
        sl = pl.ds(c * _CHUNK, _CHUNK)

        def fire(d, carry2):
            pltpu.async_copy(
                tab_hbm.at[pl.ds(d * _STRIDE, _STRIDE)].at[idx_v.at[sl]],
                dst_v.at[d, sl], sem)
            return carry2

        lax.fori_loop(0, EMBED_DIM, fire, 0)
        return carry

    lax.fori_loop(0, _N_CHUNKS, do_chunk, 0)

    pltpu.make_async_copy(out_hbm.at[:, pl.ds(0, _B_PER_W)], dst_v,
                          sem).wait()
    pltpu.sync_copy(dst_v, out_hbm.at[:, pl.ds(base, _B_PER_W)])


def kernel(user, table):
    mesh = plsc.VectorSubcoreMesh(core_axis_name="c", subcore_axis_name="s")
    run = functools.partial(
        pl.kernel,
        mesh=mesh,
        compiler_params=pltpu.CompilerParams(
            use_tc_tiling_on_sc=False,
            disable_bounds_checks=True,
            disable_semaphore_checks=True,
        ),
        out_type=jax.ShapeDtypeStruct((EMBED_DIM, BATCH), jnp.float32),
        scratch_types=[
            pltpu.VMEM((_B_PER_W,), jnp.int32),
            pltpu.VMEM((EMBED_DIM, _B_PER_W), jnp.float32),
            pltpu.SemaphoreType.DMA,
        ],
    )(_lookup_kernel)
    tab_flat = jnp.pad(table.T, ((0, 0), (0, _STRIDE - (VOCAB + 1)))
                       ).reshape(-1)
    out_t = run(user.astype(jnp.int32), tab_flat)
    return out_t.T


# speedup vs baseline: 1.0010x; 1.0010x over previous
import functools

import jax
import jax.numpy as jnp
from jax import lax
from jax.experimental import pallas as pl
from jax.experimental.pallas import tpu as pltpu
from jax.experimental.pallas import tpu_sc as plsc

VOCAB = 100000
EMBED_DIM = 16
BATCH = 16384

_NC = 2
_NS = 16
_NW = _NC * _NS
_LANES = 16

_CHUNK = 128
_B_PER_W = BATCH // _NW
_N_CHUNKS = _B_PER_W // _CHUNK
_STRIDE = VOCAB + 8


def _lookup_kernel(idx_hbm, tab_hbm, out_hbm, idx_v, dst_v, sem):
    wid = lax.axis_index("s") * _NC + lax.axis_index("c")
    base = wid * _B_PER_W

    pltpu.sync_copy(idx_hbm.at[pl.ds(base, _B_PER_W)], idx_v)

    def remap(i, carry):
        sl = pl.ds(i * _LANES, _LANES)
        v = idx_v[sl]
        idx_v[sl] = jnp.where((v >= 0) & (v < VOCAB), v + 1, 0)
        return carry

    lax.fori_loop(0, _B_PER_W // _LANES, remap, 0)

    def do_chunk(c, carry):
        sl = pl.ds(c * _CHUNK, _CHUNK)

        def fire(d, carry2):
            pltpu.async_copy(
                tab_hbm.at[pl.ds(d * _STRIDE, _STRIDE)].at[idx_v.at[sl]],
                dst_v.at[d, sl], sem)
            return carry2

        lax.fori_loop(0, EMBED_DIM, fire, 0)
        return carry

    lax.fori_loop(0, _N_CHUNKS, do_chunk, 0)

    pltpu.make_async_copy(out_hbm.at[:, pl.ds(0, _B_PER_W)], dst_v,
                          sem).wait()
    pltpu.sync_copy(dst_v, out_hbm.at[:, pl.ds(base, _B_PER_W)])


def kernel(user, table):
    mesh = plsc.VectorSubcoreMesh(core_axis_name="c", subcore_axis_name="s")
    run = functools.partial(
        pl.kernel,
        mesh=mesh,
        compiler_params=pltpu.CompilerParams(
            use_tc_tiling_on_sc=False,
            disable_bounds_checks=True,
            disable_semaphore_checks=True,
        ),
        out_type=jax.ShapeDtypeStruct((EMBED_DIM, BATCH), jnp.float32),
        scratch_types=[
            pltpu.VMEM((_B_PER_W,), jnp.int32),
            pltpu.VMEM((EMBED_DIM, _B_PER_W), jnp.float32),
            pltpu.SemaphoreType.DMA,
        ],
    )(_lookup_kernel)
    pad_cols = jnp.zeros((EMBED_DIM, _STRIDE - (VOCAB + 1)), jnp.float32)
    tab_flat = jnp.concatenate([table.T, pad_cols], axis=1).reshape(-1)
    out_t = run(user.astype(jnp.int32), tab_flat)
    return out_t.T


# speedup vs baseline: 1.1214x; 1.1203x over previous
import functools

import jax
import jax.numpy as jnp
from jax import lax
from jax.experimental import pallas as pl
from jax.experimental.pallas import tpu as pltpu
from jax.experimental.pallas import tpu_sc as plsc

VOCAB = 100000
EMBED_DIM = 16
BATCH = 16384

_NC = 2
_NS = 16
_NW = _NC * _NS
_LANES = 16

_CHUNK = 128
_B_PER_W = BATCH // _NW
_N_CHUNKS = _B_PER_W // _CHUNK
_STRIDE = VOCAB + 1
_SLICE = VOCAB + 8


def _lookup_kernel(idx_hbm, tab_hbm, out_hbm, idx_v, dst_v, sem):
    wid = lax.axis_index("s") * _NC + lax.axis_index("c")
    base = wid * _B_PER_W

    pltpu.sync_copy(idx_hbm.at[pl.ds(base, _B_PER_W)], idx_v.at[0])

    def remap(i, carry):
        sl = pl.ds(i * _LANES, _LANES)
        v = idx_v[0, sl]
        idx_v[0, sl] = jnp.where((v >= 0) & (v < VOCAB), v + 1, 0)
        return carry

    lax.fori_loop(0, _B_PER_W // _LANES, remap, 0)

    def shift(i, carry):
        r = i // (_B_PER_W // _LANES) + 1
        sl = pl.ds((i % (_B_PER_W // _LANES)) * _LANES, _LANES)
        idx_v[r, sl] = idx_v[0, sl] + r
        return carry

    lax.fori_loop(0, 7 * (_B_PER_W // _LANES), shift, 0)

    def do_chunk(c, carry):
        sl = pl.ds(c * _CHUNK, _CHUNK)
        for d in range(EMBED_DIM):
            pltpu.async_copy(
                tab_hbm.at[pl.ds(d * _STRIDE - d % 8, _SLICE)]
                .at[idx_v.at[d % 8, sl]],
                dst_v.at[d, sl], sem)
        return carry

    lax.fori_loop(0, _N_CHUNKS, do_chunk, 0)

    pltpu.make_async_copy(out_hbm.at[:, pl.ds(0, _B_PER_W)], dst_v,
                          sem).wait()
    pltpu.sync_copy(dst_v, out_hbm.at[:, pl.ds(base, _B_PER_W)])


def kernel(user, table):
    mesh = plsc.VectorSubcoreMesh(core_axis_name="c", subcore_axis_name="s")
    run = functools.partial(
        pl.kernel,
        mesh=mesh,
        compiler_params=pltpu.CompilerParams(
            use_tc_tiling_on_sc=False,
            disable_bounds_checks=True,
            disable_semaphore_checks=True,
        ),
        out_type=jax.ShapeDtypeStruct((EMBED_DIM, BATCH), jnp.float32),
        scratch_types=[
            pltpu.VMEM((8, _B_PER_W), jnp.int32),
            pltpu.VMEM((EMBED_DIM, _B_PER_W), jnp.float32),
            pltpu.SemaphoreType.DMA,
        ],
    )(_lookup_kernel)
    tab_flat = table.T.reshape(-1)
    out_t = run(user.astype(jnp.int32), tab_flat)
    return out_t.T
